# baseline (speedup 1.0000x reference)
import functools

import jax
import jax.numpy as jnp
from jax import lax
from jax.experimental import pallas as pl
from jax.experimental.pallas import tpu as pltpu
from jax.experimental.pallas import tpu_sc as plsc

_NUM_GENES = 100000
_EMBED_DIM = 32
_BATCH = 16384

_CHUNK = 4096
_LANES = 16


def _build():
    info = plsc.get_sparse_core_info()
    nw = info.num_cores * info.num_subcores
    n_chunks = _BATCH // _CHUNK
    n_groups = _CHUNK // _LANES
    mesh = plsc.VectorSubcoreMesh(core_axis_name="c", subcore_axis_name="s")

    @functools.partial(
        pl.kernel,
        mesh=mesh,
        out_type=jax.ShapeDtypeStruct((_EMBED_DIM, _BATCH), jnp.float32),
        scratch_types=[
            pltpu.VMEM((_NUM_GENES,), jnp.float32),
            pltpu.VMEM((_BATCH,), jnp.int32),
            pltpu.VMEM((_CHUNK,), jnp.float32),
            pltpu.VMEM((_CHUNK,), jnp.float32),
            pltpu.SemaphoreType.DMA,
            pltpu.SemaphoreType.DMA,
            pltpu.SemaphoreType.DMA,
        ],
        compiler_params=pltpu.CompilerParams(needs_layout_passes=False),
    )
    def gather_kernel(
        tab_t_hbm, idx_hbm, out_hbm, row_v, idx_v, out_a, out_b, sem_r, sem_a, sem_b
    ):
        w = lax.axis_index("s") * info.num_cores + lax.axis_index("c")
        row_cp = pltpu.async_copy(tab_t_hbm.at[w], row_v, sem_r)
        idx_cp = pltpu.async_copy(idx_hbm, idx_v, sem_a)
        idx_cp.wait()
        row_cp.wait()

        sems = (sem_a, sem_b)
        bufs = (out_a, out_b)
        stores = [None, None]
        for c in range(n_chunks):
            buf = bufs[c % 2]
            if stores[c % 2] is not None:
                stores[c % 2].wait()

            idx_c = idx_v.at[pl.ds(c * _CHUNK, _CHUNK)]

            @plsc.parallel_loop(0, _CHUNK, step=_LANES, unroll=8)
            def _(i, buf=buf, idx_c=idx_c):
                s = pl.ds(i, _LANES)
                buf[s] = plsc.load_gather(row_v, [idx_c[s]])

            stores[c % 2] = pltpu.async_copy(
                buf, out_hbm.at[w, pl.ds(c * _CHUNK, _CHUNK)], sems[c % 2]
            )
        stores[0].wait()
        stores[1].wait()

    return gather_kernel


def kernel(gene_idx, embedding_table):
    gather = _build()
    out_t = gather(embedding_table.T, gene_idx.astype(jnp.int32))
    return out_t.T


# speedup vs baseline: 1.0021x; 1.0021x over previous
import functools

import jax
import jax.numpy as jnp
from jax import lax
from jax.experimental import pallas as pl
from jax.experimental.pallas import tpu as pltpu
from jax.experimental.pallas import tpu_sc as plsc

_NUM_GENES = 100000
_EMBED_DIM = 32
_BATCH = 16384

_CHUNK = 4096
_LANES = 16


def _build():
    info = plsc.get_sparse_core_info()
    nw = info.num_cores * info.num_subcores
    assert nw == _EMBED_DIM
    n_chunks = _BATCH // _CHUNK
    n_groups = _CHUNK // _LANES
    mesh = plsc.VectorSubcoreMesh(core_axis_name="c", subcore_axis_name="s")

    @functools.partial(
        pl.kernel,
        mesh=mesh,
        out_type=jax.ShapeDtypeStruct((_EMBED_DIM, _BATCH), jnp.float32),
        scratch_types=[
            pltpu.VMEM((_NUM_GENES,), jnp.float32),
            pltpu.VMEM((_BATCH,), jnp.int32),
            pltpu.VMEM((_CHUNK,), jnp.float32),
            pltpu.VMEM((_CHUNK,), jnp.float32),
            pltpu.SemaphoreType.DMA,
            pltpu.SemaphoreType.DMA,
            pltpu.SemaphoreType.DMA,
        ],
        compiler_params=pltpu.CompilerParams(needs_layout_passes=False),
    )
    def gather_kernel(
        tab_t_hbm, idx_hbm, out_hbm, row_v, idx_v, out_a, out_b, sem_r, sem_a, sem_b
    ):
        w = lax.axis_index("s") * info.num_cores + lax.axis_index("c")
        row_cp = pltpu.async_copy(tab_t_hbm.at[w], row_v, sem_r)
        idx_cp = pltpu.async_copy(idx_hbm, idx_v, sem_a)
        idx_cp.wait()
        row_cp.wait()

        sems = (sem_a, sem_b)
        bufs = (out_a, out_b)
        stores = [None, None]
        for c in range(n_chunks):
            buf = bufs[c % 2]
            if stores[c % 2] is not None:
                stores[c % 2].wait()

            idx_c = idx_v.at[pl.ds(c * _CHUNK, _CHUNK)]

            @plsc.parallel_loop(0, _CHUNK, step=_LANES, unroll=8)
            def _(i, buf=buf, idx_c=idx_c):
                s = pl.ds(i, _LANES)
                buf[s] = plsc.load_gather(row_v, [idx_c[s]])

            stores[c % 2] = pltpu.async_copy(
                buf, out_hbm.at[w, pl.ds(c * _CHUNK, _CHUNK)], sems[c % 2]
            )
        stores[0].wait()
        stores[1].wait()

    return gather_kernel


def kernel(gene_idx, embedding_table):
    gather = _build()
    out_t = gather(embedding_table.T, gene_idx.astype(jnp.int32))
    return out_t.T
